# baseline (speedup 1.0000x reference)
import functools

import jax
import jax.numpy as jnp
from jax import lax
from jax.experimental import pallas as pl
from jax.experimental.pallas import tpu as pltpu
from jax.experimental.pallas import tpu_sc as plsc

D = 64
SCALE = 8.0
NC, NS, L = 2, 16, 16
NW = NC * NS
BATCH = 4096
SEQ = 200
BL = 128
EH, ELO = 8, 8


def _emb_body(toks_hbm, table_hbm, out_hbm,
              idx_v, in0, in1, tb0, tb1, g0, g1, s0, s1):
  wid = lax.axis_index("s") * NC + lax.axis_index("c")

  pltpu.sync_copy(toks_hbm.at[wid], idx_v)

  ins = (in0, in1)
  tbs = (tb0, tb1)
  gsems = (g0, g1)
  ssems = (s0, s1)

  ridx = [lax.iota(jnp.int32, L) + blk * L for blk in range(BL // L)]

  def start_gather(t, b):
    pltpu.make_async_copy(table_hbm.at[idx_v.at[t]], ins[b], gsems[b]).start()

  def wait_gather(t, b):
    pltpu.make_async_copy(table_hbm.at[idx_v.at[t]], ins[b], gsems[b]).wait()

  def start_store(t, b):
    for eh in range(EH):
      pltpu.make_async_copy(
          tbs[b].at[eh], out_hbm.at[t, eh, wid], ssems[b]).start()

  def wait_store(b):
    for eh in range(EH):
      pltpu.make_async_copy(
          tbs[b].at[eh], out_hbm.at[0, eh, wid], ssems[b]).wait()

  def transpose_scale(b):
    ib = ins[b]
    tb = tbs[b]

    def col(e, carry):
      eh = e // ELO
      el = e % ELO
      cidx = jnp.full((L,), 0, jnp.int32) + e
      for blk in range(BL // L):
        vals = plsc.load_gather(ib, [ridx[blk], cidx])
        tb[eh, el, pl.ds(blk * L, L)] = vals * SCALE
      return carry

    lax.fori_loop(0, D, col, 0)

  start_gather(0, 0)
  start_gather(1, 1)

  for t in (0, 1):
    b = t
    wait_gather(t, b)
    transpose_scale(b)
    start_gather(t + 2, b)
    start_store(t, b)

  def mid(k, carry):
    for b in range(2):
      t = 2 * k + b
      wait_gather(t, b)
      wait_store(b)
      transpose_scale(b)
      start_gather(t + 2, b)
      start_store(t, b)
    return carry

  lax.fori_loop(1, SEQ // 2 - 1, mid, 0)

  for t in (SEQ - 2, SEQ - 1):
    b = t % 2
    wait_gather(t, b)
    wait_store(b)
    transpose_scale(b)
    start_store(t, b)

  for b in range(2):
    wait_store(b)


_emb_call = functools.partial(
    pl.kernel,
    out_type=jax.ShapeDtypeStruct((SEQ, EH, NW, ELO, BL), jnp.float32),
    mesh=plsc.VectorSubcoreMesh(core_axis_name="c", subcore_axis_name="s"),
    compiler_params=pltpu.CompilerParams(
        use_tc_tiling_on_sc=False, needs_layout_passes=False),
    scratch_types=[
        pltpu.VMEM((SEQ, BL), jnp.int32),
        pltpu.VMEM((BL, D), jnp.float32),
        pltpu.VMEM((BL, D), jnp.float32),
        pltpu.VMEM((EH, ELO, BL), jnp.float32),
        pltpu.VMEM((EH, ELO, BL), jnp.float32),
        pltpu.SemaphoreType.DMA,
        pltpu.SemaphoreType.DMA,
        pltpu.SemaphoreType.DMA,
        pltpu.SemaphoreType.DMA,
    ],
)(_emb_body)


@jax.jit
def kernel(tokens, table):
  toks3 = (tokens.astype(jnp.int32).T.reshape(SEQ, NW, BL)
           .transpose(1, 0, 2)) * 2
  table2 = jnp.pad(table, ((0, 0), (0, D))).reshape(2 * table.shape[0], D)
  out5 = _emb_call(toks3, table2)
  return out5.transpose(2, 4, 0, 1, 3).reshape(BATCH, SEQ, D)


# speedup vs baseline: 1.7431x; 1.7431x over previous
import functools

import jax
import jax.numpy as jnp
from jax import lax
from jax.experimental import pallas as pl
from jax.experimental.pallas import tpu as pltpu
from jax.experimental.pallas import tpu_sc as plsc

D = 64
SCALE = 8.0
NC, NS, L = 2, 16, 16
NW = NC * NS
BATCH = 4096
SEQ = 200
BL = 128
EH, ELO = 8, 8


def _emb_body(toks_hbm, table_hbm, out_hbm,
              idx_v, in0, in1, tb0, tb1, g0, g1, s0, s1):
  wid = lax.axis_index("s") * NC + lax.axis_index("c")

  pltpu.sync_copy(toks_hbm.at[wid], idx_v)

  ins = (in0, in1)
  tbs = (tb0, tb1)
  gsems = (g0, g1)
  ssems = (s0, s1)

  lanes = lax.iota(jnp.int32, L)
  ridx = [lanes + blk * L for blk in range(BL // L)]

  def start_gather(t, b):
    pltpu.make_async_copy(table_hbm.at[idx_v.at[t]], ins[b], gsems[b]).start()

  def wait_gather(t, b):
    pltpu.make_async_copy(table_hbm.at[idx_v.at[t]], ins[b], gsems[b]).wait()

  def start_store(t, b):
    for eh in range(EH):
      pltpu.make_async_copy(
          tbs[b].at[eh], out_hbm.at[t, eh, wid], ssems[b]).start()

  def wait_store(b):
    for eh in range(EH):
      pltpu.make_async_copy(
          tbs[b].at[eh], out_hbm.at[0, eh, wid], ssems[b]).wait()

  def transpose_scale(b):
    ib = ins[b]
    tb = tbs[b]

    def col(c, carry):
      cm = (c + lanes) & (D - 1)
      eh = cm >> 3
      el = cm & (ELO - 1)
      for blk in range(BL // L):
        vals = plsc.load_gather(ib, [ridx[blk], cm])
        plsc.store_scatter(tb, [eh, el, ridx[blk]], vals * SCALE)
      return carry

    lax.fori_loop(0, D, col, 0)

  start_gather(0, 0)
  start_gather(1, 1)

  for t in (0, 1):
    b = t
    wait_gather(t, b)
    transpose_scale(b)
    start_gather(t + 2, b)
    start_store(t, b)

  def mid(k, carry):
    for b in range(2):
      t = 2 * k + b
      wait_gather(t, b)
      wait_store(b)
      transpose_scale(b)
      start_gather(t + 2, b)
      start_store(t, b)
    return carry

  lax.fori_loop(1, SEQ // 2 - 1, mid, 0)

  for t in (SEQ - 2, SEQ - 1):
    b = t % 2
    wait_gather(t, b)
    wait_store(b)
    transpose_scale(b)
    start_store(t, b)

  for b in range(2):
    wait_store(b)


_emb_call = functools.partial(
    pl.kernel,
    out_type=jax.ShapeDtypeStruct((SEQ, EH, NW, ELO, BL), jnp.float32),
    mesh=plsc.VectorSubcoreMesh(core_axis_name="c", subcore_axis_name="s"),
    compiler_params=pltpu.CompilerParams(
        use_tc_tiling_on_sc=False, needs_layout_passes=False),
    scratch_types=[
        pltpu.VMEM((SEQ, BL), jnp.int32),
        pltpu.VMEM((BL, D), jnp.float32),
        pltpu.VMEM((BL, D), jnp.float32),
        pltpu.VMEM((EH, ELO, BL), jnp.float32),
        pltpu.VMEM((EH, ELO, BL), jnp.float32),
        pltpu.SemaphoreType.DMA,
        pltpu.SemaphoreType.DMA,
        pltpu.SemaphoreType.DMA,
        pltpu.SemaphoreType.DMA,
    ],
)(_emb_body)


@jax.jit
def kernel(tokens, table):
  toks3 = (tokens.astype(jnp.int32).T.reshape(SEQ, NW, BL)
           .transpose(1, 0, 2)) * 2
  table2 = jnp.pad(table, ((0, 0), (0, D))).reshape(2 * table.shape[0], D)
  out5 = _emb_call(toks3, table2)
  return out5.transpose(2, 4, 0, 1, 3).reshape(BATCH, SEQ, D)


# speedup vs baseline: 2.5155x; 1.4431x over previous
import functools

import jax
import jax.numpy as jnp
from jax import lax
from jax.experimental import pallas as pl
from jax.experimental.pallas import tpu as pltpu
from jax.experimental.pallas import tpu_sc as plsc

D = 64
SCALE = 8.0
NC, NS, L = 2, 16, 16
NW = NC * NS
BATCH = 4096
SEQ = 200
BL = 128
EH, ELO = 8, 8


def _emb_body(toks_hbm, table_hbm, out_hbm,
              idx_v, in0, in1, tb0, tb1, g0, g1, s0, s1):
  wid = lax.axis_index("s") * NC + lax.axis_index("c")

  pltpu.sync_copy(toks_hbm.at[wid], idx_v)

  ins = (in0, in1)
  tbs = (tb0, tb1)
  gsems = (g0, g1)
  ssems = (s0, s1)

  lanes = lax.iota(jnp.int32, L)
  ridx = [lanes + blk * L for blk in range(BL // L)]

  def start_gather(t, b):
    pltpu.make_async_copy(table_hbm.at[idx_v.at[t]], ins[b], gsems[b]).start()

  def wait_gather(t, b):
    pltpu.make_async_copy(table_hbm.at[idx_v.at[t]], ins[b], gsems[b]).wait()

  def start_store(t, b):
    for eh in range(EH):
      pltpu.make_async_copy(
          tbs[b].at[eh], out_hbm.at[t, eh, wid], ssems[b]).start()

  def wait_store(b):
    for eh in range(EH):
      pltpu.make_async_copy(
          tbs[b].at[eh], out_hbm.at[0, eh, wid], ssems[b]).wait()

  def transpose_scale(b):
    ib = ins[b]
    tb = tbs[b]

    @plsc.parallel_loop(0, D, unroll=4)
    def col(c):
      cm = (c + lanes) & (D - 1)
      eh = cm >> 3
      el = cm & (ELO - 1)
      for blk in range(BL // L):
        vals = plsc.load_gather(ib, [ridx[blk], cm])
        plsc.store_scatter(tb, [eh, el, ridx[blk]], vals * SCALE)

  start_gather(0, 0)
  start_gather(1, 1)

  for t in (0, 1):
    b = t
    wait_gather(t, b)
    transpose_scale(b)
    start_gather(t + 2, b)
    start_store(t, b)

  def mid(k, carry):
    for b in range(2):
      t = 2 * k + b
      wait_gather(t, b)
      wait_store(b)
      transpose_scale(b)
      start_gather(t + 2, b)
      start_store(t, b)
    return carry

  lax.fori_loop(1, SEQ // 2 - 1, mid, 0)

  for t in (SEQ - 2, SEQ - 1):
    b = t % 2
    wait_gather(t, b)
    wait_store(b)
    transpose_scale(b)
    start_store(t, b)

  for b in range(2):
    wait_store(b)


_emb_call = functools.partial(
    pl.kernel,
    out_type=jax.ShapeDtypeStruct((SEQ, EH, NW, ELO, BL), jnp.float32),
    mesh=plsc.VectorSubcoreMesh(core_axis_name="c", subcore_axis_name="s"),
    compiler_params=pltpu.CompilerParams(
        use_tc_tiling_on_sc=False, needs_layout_passes=False),
    scratch_types=[
        pltpu.VMEM((SEQ, BL), jnp.int32),
        pltpu.VMEM((BL, D), jnp.float32),
        pltpu.VMEM((BL, D), jnp.float32),
        pltpu.VMEM((EH, ELO, BL), jnp.float32),
        pltpu.VMEM((EH, ELO, BL), jnp.float32),
        pltpu.SemaphoreType.DMA,
        pltpu.SemaphoreType.DMA,
        pltpu.SemaphoreType.DMA,
        pltpu.SemaphoreType.DMA,
    ],
)(_emb_body)


@jax.jit
def kernel(tokens, table):
  toks3 = (tokens.astype(jnp.int32).T.reshape(SEQ, NW, BL)
           .transpose(1, 0, 2)) * 2
  table2 = jnp.pad(table, ((0, 0), (0, D))).reshape(2 * table.shape[0], D)
  out5 = _emb_call(toks3, table2)
  return out5.transpose(2, 4, 0, 1, 3).reshape(BATCH, SEQ, D)
